# baseline (speedup 1.0000x reference)
import functools

import jax
import jax.numpy as jnp
from jax.experimental import pallas as pl
from jax.experimental.pallas import tpu as pltpu

B, L, H, E = 4, 4096, 2048, 16
CHUNK = 512
NL = L // CHUNK


def _body(h_ref, m_ref, w1_ref, b1_ref, w2_ref, b2_ref,
          seqw_ref, exp_ref, acc_ref, len_ref):
    b = pl.program_id(0)
    j = pl.program_id(1)

    mask = m_ref[pl.ds(b, 1), pl.ds(j * CHUNK, CHUNK)][0, :]
    hblk = h_ref[0, :, :]
    part = jnp.sum(hblk * mask[:, None], axis=0)
    plen = jnp.sum(mask)

    @pl.when(j == 0)
    def _init():
        acc_ref[0, :] = part
        len_ref[0] = plen

    @pl.when(j > 0)
    def _acc():
        acc_ref[0, :] = acc_ref[0, :] + part
        len_ref[0] = len_ref[0] + plen

    @pl.when(j == NL - 1)
    def _finish():
        length = jnp.maximum(len_ref[0], 1.0)
        pooled = (acc_ref[0:1, :] / length)
        hmid = jnp.maximum(
            jnp.dot(pooled, w1_ref[:, :], preferred_element_type=jnp.float32)
            + b1_ref[0:1, :], 0.0)
        logits = (jnp.dot(hmid, w2_ref[:, :], preferred_element_type=jnp.float32)
                  + b2_ref[0:1, :])

        idx = jax.lax.broadcasted_iota(jnp.int32, (1, E), 1)
        m1 = jnp.max(logits, axis=1, keepdims=True)
        i1 = jnp.min(jnp.where(logits == m1, idx, E), axis=1, keepdims=True)
        masked = jnp.where(idx == i1, -jnp.inf, logits)
        m2 = jnp.max(masked, axis=1, keepdims=True)
        i2 = jnp.min(jnp.where(masked == m2, idx, E), axis=1, keepdims=True)

        e2 = jnp.exp(m2 - m1)
        denom = 1.0 + e2
        w_top = 1.0 / denom
        w_snd = e2 / denom
        seqw = jnp.where(idx == i1, w_top,
                         jnp.where(idx == i2, w_snd, 0.0))
        seqw_ref[pl.ds(b, 1), :] = seqw
        exp_ref[0, :, :] = jnp.broadcast_to(seqw, (L, E))


@jax.jit
def kernel(hidden_states, attention_mask, W1, b1, W2, b2):
    seqw, expanded = pl.pallas_call(
        _body,
        grid=(B, NL),
        in_specs=[
            pl.BlockSpec((1, CHUNK, H), lambda b, j: (b, j, 0)),
            pl.BlockSpec((B, L), lambda b, j: (0, 0)),
            pl.BlockSpec((H, H // 2), lambda b, j: (0, 0)),
            pl.BlockSpec((1, H // 2), lambda b, j: (0, 0)),
            pl.BlockSpec((H // 2, E), lambda b, j: (0, 0)),
            pl.BlockSpec((1, E), lambda b, j: (0, 0)),
        ],
        out_specs=[
            pl.BlockSpec((B, E), lambda b, j: (0, 0)),
            pl.BlockSpec((1, L, E), lambda b, j: (b, 0, 0)),
        ],
        out_shape=[
            jax.ShapeDtypeStruct((B, E), jnp.float32),
            jax.ShapeDtypeStruct((B, L, E), jnp.float32),
        ],
        scratch_shapes=[
            pltpu.VMEM((8, H), jnp.float32),
            pltpu.SMEM((1,), jnp.float32),
        ],
        compiler_params=pltpu.CompilerParams(
            dimension_semantics=("arbitrary", "arbitrary"),
        ),
    )(hidden_states, attention_mask, W1,
      b1.reshape(1, H // 2), W2, b2.reshape(1, E))
    return seqw, expanded


# speedup vs baseline: 1.1171x; 1.1171x over previous
import jax
import jax.numpy as jnp
from jax.experimental import pallas as pl
from jax.experimental.pallas import tpu as pltpu

B, L, H, E = 4, 4096, 2048, 16
CHUNK = 512
NL = L // CHUNK


def _body(h_ref, m_ref, w1_ref, b1_ref, w2_ref, b2_ref,
          seqw_ref, exp_ref, acc_ref):
    j = pl.program_id(0)

    mask = m_ref[:, pl.ds(j * CHUNK, CHUNK)]
    part = jnp.sum(h_ref[:, :, :] * mask[:, :, None], axis=1)

    @pl.when(j == 0)
    def _init():
        acc_ref[0:B, :] = part

    @pl.when(j > 0)
    def _acc():
        acc_ref[0:B, :] = acc_ref[0:B, :] + part

    @pl.when(j == NL - 1)
    def _finish():
        lengths = jnp.sum(m_ref[:, :], axis=1, keepdims=True)
        pooled = acc_ref[0:B, :] / jnp.maximum(lengths, 1.0)
        hmid = jnp.maximum(
            jnp.dot(pooled, w1_ref[:, :], preferred_element_type=jnp.float32)
            + b1_ref[0:1, :], 0.0)
        logits = (jnp.dot(hmid, w2_ref[:, :], preferred_element_type=jnp.float32)
                  + b2_ref[0:1, :])

        idx = jax.lax.broadcasted_iota(jnp.int32, (B, E), 1)
        m1 = jnp.max(logits, axis=1, keepdims=True)
        i1 = jnp.min(jnp.where(logits == m1, idx, E), axis=1, keepdims=True)
        masked = jnp.where(idx == i1, -jnp.inf, logits)
        m2 = jnp.max(masked, axis=1, keepdims=True)
        i2 = jnp.min(jnp.where(masked == m2, idx, E), axis=1, keepdims=True)

        e2 = jnp.exp(m2 - m1)
        w_top = 1.0 / (1.0 + e2)
        w_snd = e2 / (1.0 + e2)
        seqw = jnp.where(idx == i1, w_top,
                         jnp.where(idx == i2, w_snd, 0.0))
        seqw_ref[:, :] = seqw
        exp_ref[:, :, :] = jnp.broadcast_to(seqw[:, None, :], (B, L, E))


@jax.jit
def kernel(hidden_states, attention_mask, W1, b1, W2, b2):
    seqw, expanded = pl.pallas_call(
        _body,
        grid=(NL,),
        in_specs=[
            pl.BlockSpec((B, CHUNK, H), lambda j: (0, j, 0)),
            pl.BlockSpec((B, L), lambda j: (0, 0)),
            pl.BlockSpec((H, H // 2), lambda j: (0, 0)),
            pl.BlockSpec((1, H // 2), lambda j: (0, 0)),
            pl.BlockSpec((H // 2, E), lambda j: (0, 0)),
            pl.BlockSpec((1, E), lambda j: (0, 0)),
        ],
        out_specs=[
            pl.BlockSpec((B, E), lambda j: (0, 0)),
            pl.BlockSpec((B, L, E), lambda j: (0, 0, 0)),
        ],
        out_shape=[
            jax.ShapeDtypeStruct((B, E), jnp.float32),
            jax.ShapeDtypeStruct((B, L, E), jnp.float32),
        ],
        scratch_shapes=[
            pltpu.VMEM((8, H), jnp.float32),
        ],
        compiler_params=pltpu.CompilerParams(
            dimension_semantics=("arbitrary",),
        ),
    )(hidden_states, attention_mask, W1,
      b1.reshape(1, H // 2), W2, b2.reshape(1, E))
    return seqw, expanded
